# Initial kernel scaffold; baseline (speedup 1.0000x reference)
import jax
import jax.numpy as jnp
from jax.experimental import pallas as pl


def kernel(word, pos0, pos1, pos2, word_table, pos0_table, pos1_table, pos2_table):
    raise NotImplementedError("write your pallas kernel here")


# speedup vs baseline: 5.1463x; 5.1463x over previous
import functools

import jax
import jax.numpy as jnp
from jax import lax
from jax.experimental import pallas as pl
from jax.experimental.pallas import tpu as pltpu
from jax.experimental.pallas import tpu_sc as plsc

WORD_DIM = 128
POS_DIM = 16
OUT_DIM = WORD_DIM + 3 * POS_DIM
NUM_CORES = 2
NUM_SUBCORES = 16
NUM_WORKERS = NUM_CORES * NUM_SUBCORES
CHUNK = 128


def _build(BL: int):
    rows_per_w = BL // NUM_WORKERS
    n_chunks = rows_per_w // CHUNK
    mesh = plsc.VectorSubcoreMesh(
        core_axis_name="c", subcore_axis_name="s",
        num_cores=NUM_CORES, num_subcores=NUM_SUBCORES)

    @functools.partial(
        pl.kernel,
        mesh=mesh,
        compiler_params=pltpu.CompilerParams(use_tc_tiling_on_sc=False),
        out_type=jax.ShapeDtypeStruct((BL, OUT_DIM), jnp.float32),
        scratch_types=[
            pltpu.VMEM((CHUNK,), jnp.int32),
            pltpu.VMEM((CHUNK,), jnp.int32),
            pltpu.VMEM((CHUNK,), jnp.int32),
            pltpu.VMEM((CHUNK,), jnp.int32),
            pltpu.VMEM((CHUNK, WORD_DIM), jnp.float32),
            pltpu.VMEM((CHUNK, POS_DIM), jnp.float32),
            pltpu.VMEM((CHUNK, POS_DIM), jnp.float32),
            pltpu.VMEM((CHUNK, POS_DIM), jnp.float32),
            pltpu.SemaphoreType.DMA,
            pltpu.SemaphoreType.DMA,
        ],
    )
    def fused_embed(word_h, p0_h, p1_h, p2_h, wt_h, t0_h, t1_h, t2_h,
                    out_h, iw, i0, i1, i2, bw, b0, b1, b2, sem_i, sem_g):
        wid = lax.axis_index("s") * NUM_CORES + lax.axis_index("c")
        base0 = wid * rows_per_w

        def step(g, carry):
            base = base0 + g * CHUNK
            ci = (pltpu.async_copy(word_h.at[pl.ds(base, CHUNK)], iw, sem_i),
                  pltpu.async_copy(p0_h.at[pl.ds(base, CHUNK)], i0, sem_i),
                  pltpu.async_copy(p1_h.at[pl.ds(base, CHUNK)], i1, sem_i),
                  pltpu.async_copy(p2_h.at[pl.ds(base, CHUNK)], i2, sem_i))
            for c in ci:
                c.wait()
            cg = (pltpu.async_copy(wt_h.at[iw], bw, sem_g),
                  pltpu.async_copy(t0_h.at[i0], b0, sem_g),
                  pltpu.async_copy(t1_h.at[i1], b1, sem_g),
                  pltpu.async_copy(t2_h.at[i2], b2, sem_g))
            for c in cg:
                c.wait()
            rows = pl.ds(base, CHUNK)
            pltpu.sync_copy(bw, out_h.at[rows, pl.ds(0, WORD_DIM)])
            pltpu.sync_copy(b0, out_h.at[rows, pl.ds(WORD_DIM, POS_DIM)])
            pltpu.sync_copy(b1, out_h.at[rows, pl.ds(WORD_DIM + POS_DIM, POS_DIM)])
            pltpu.sync_copy(b2, out_h.at[rows, pl.ds(WORD_DIM + 2 * POS_DIM, POS_DIM)])
            return carry

        lax.fori_loop(0, n_chunks, step, 0)

    return fused_embed


def kernel(word, pos0, pos1, pos2, word_table, pos0_table, pos1_table, pos2_table):
    B, L = word.shape
    BL = B * L
    fused = _build(BL)
    out = fused(word.reshape(BL), pos0.reshape(BL), pos1.reshape(BL),
                pos2.reshape(BL), word_table, pos0_table, pos1_table,
                pos2_table)
    return out.reshape(B, L, OUT_DIM)


# speedup vs baseline: 5.5724x; 1.0828x over previous
import functools

import jax
import jax.numpy as jnp
from jax import lax
from jax.experimental import pallas as pl
from jax.experimental.pallas import tpu as pltpu
from jax.experimental.pallas import tpu_sc as plsc

WORD_DIM = 128
POS_DIM = 16
OUT_DIM = WORD_DIM + 3 * POS_DIM
NUM_CORES = 2
NUM_SUBCORES = 16
NUM_WORKERS = NUM_CORES * NUM_SUBCORES
CHUNK = 128
NBUF = 4

_DIMS = (WORD_DIM, POS_DIM, POS_DIM, POS_DIM)
_OFFS = (0, WORD_DIM, WORD_DIM + POS_DIM, WORD_DIM + 2 * POS_DIM)


def _build(BL: int):
    rows_per_w = BL // NUM_WORKERS
    n_chunks = rows_per_w // CHUNK
    n_iters = n_chunks // NBUF
    mesh = plsc.VectorSubcoreMesh(
        core_axis_name="c", subcore_axis_name="s",
        num_cores=NUM_CORES, num_subcores=NUM_SUBCORES)

    scratch = (
        [pltpu.VMEM((CHUNK,), jnp.int32) for _ in range(4 * NBUF)]
        + [pltpu.VMEM((CHUNK, d), jnp.float32) for _ in range(NBUF) for d in _DIMS]
        + [pltpu.SemaphoreType.DMA for _ in range(3 * NBUF)]
    )

    @functools.partial(
        pl.kernel,
        mesh=mesh,
        compiler_params=pltpu.CompilerParams(use_tc_tiling_on_sc=False),
        out_type=jax.ShapeDtypeStruct((BL, OUT_DIM), jnp.float32),
        scratch_types=scratch,
    )
    def fused_embed(word_h, p0_h, p1_h, p2_h, wt_h, t0_h, t1_h, t2_h,
                    out_h, *scr):
        idx_bufs = [scr[4 * k:4 * k + 4] for k in range(NBUF)]
        row_bufs = [scr[4 * NBUF + 4 * k:4 * NBUF + 4 * k + 4] for k in range(NBUF)]
        sem_i = scr[8 * NBUF:9 * NBUF]
        sem_g = scr[9 * NBUF:10 * NBUF]
        sem_o = scr[10 * NBUF:11 * NBUF]
        srcs = (word_h, p0_h, p1_h, p2_h)
        tabs = (wt_h, t0_h, t1_h, t2_h)

        wid = lax.axis_index("s") * NUM_CORES + lax.axis_index("c")
        base0 = wid * rows_per_w
        last = base0 + rows_per_w - CHUNK

        def idx_issue(g, k):
            base = jnp.minimum(base0 + g * CHUNK, last)
            for s, d in zip(srcs, idx_bufs[k]):
                pltpu.async_copy(s.at[pl.ds(base, CHUNK)], d, sem_i[k])

        def idx_wait(k):
            for s, d in zip(srcs, idx_bufs[k]):
                pltpu.make_async_copy(s.at[pl.ds(base0, CHUNK)], d, sem_i[k]).wait()

        def gather_issue(k):
            for tab, iv, b in zip(tabs, idx_bufs[k], row_bufs[k]):
                pltpu.async_copy(tab.at[iv], b, sem_g[k])

        def gather_wait(k):
            for tab, iv, b in zip(tabs, idx_bufs[k], row_bufs[k]):
                pltpu.make_async_copy(tab.at[iv], b, sem_g[k]).wait()

        def write_issue(g, k):
            base = base0 + g * CHUNK
            for b, off, d in zip(row_bufs[k], _OFFS, _DIMS):
                pltpu.async_copy(b, out_h.at[pl.ds(base, CHUNK), pl.ds(off, d)],
                                 sem_o[k])

        def write_wait(k):
            for b, off, d in zip(row_bufs[k], _OFFS, _DIMS):
                pltpu.make_async_copy(b, out_h.at[pl.ds(base0, CHUNK), pl.ds(off, d)],
                                      sem_o[k]).wait()

        for k in range(NBUF):
            idx_issue(k, k)

        def body(t, carry):
            g0 = t * NBUF
            for k in range(NBUF):
                @pl.when(t > 0)
                def _(k=k):
                    write_wait(k)
                idx_wait(k)
                gather_issue(k)
            for k in range(NBUF):
                gather_wait(k)
                write_issue(g0 + k, k)
                idx_issue(g0 + k + NBUF, k)
            return carry

        lax.fori_loop(0, n_iters, body, 0)
        for k in range(NBUF):
            idx_wait(k)
            write_wait(k)

    return fused_embed


def kernel(word, pos0, pos1, pos2, word_table, pos0_table, pos1_table, pos2_table):
    B, L = word.shape
    BL = B * L
    fused = _build(BL)
    out = fused(word.reshape(BL), pos0.reshape(BL), pos1.reshape(BL),
                pos2.reshape(BL), word_table, pos0_table, pos1_table,
                pos2_table)
    return out.reshape(B, L, OUT_DIM)
